# baseline (speedup 1.0000x reference)
import functools

import jax
import jax.numpy as jnp
from jax import lax
from jax.experimental import pallas as pl
from jax.experimental.pallas import tpu as pltpu
from jax.experimental.pallas import tpu_sc as plsc

N = 10000
E = 160000
CK = 128
NC = 2
NS = 16
EB = 125
ER = E // EB
RB = 2000

RS0 = 624
TAIL = N - NS * RS0
TAIL_OFF = N - TAIL


def _node_copy(make_src, make_dst, sid):
    pltpu.sync_copy(make_src(sid * RS0, RS0), make_dst(sid * RS0, RS0))

    @pl.when(sid == NS - 1)
    def _():
        pltpu.sync_copy(make_src(TAIL_OFF, TAIL), make_dst(TAIL_OFF, TAIL))


_IBA = ER // (NC * NS)


def _sc_degree(dst2d, zeros8, ones8):
    mesh = plsc.VectorSubcoreMesh(core_axis_name="c", subcore_axis_name="s")

    @functools.partial(
        pl.kernel,
        out_type=(
            jax.ShapeDtypeStruct((N, CK), jnp.float32),
            jax.ShapeDtypeStruct((N, CK), jnp.float32),
        ),
        mesh=mesh,
        scratch_types=[
            pltpu.VMEM((_IBA, EB), jnp.int32),
            pltpu.VMEM((EB, CK), jnp.float32),
            pltpu.VMEM_SHARED((N, CK), jnp.float32),
        ],
    )
    def deg_kernel(dst_h, zeros_h, ones_h, dega_h, degb_h, didx, ones_v, acc):
        cid = lax.axis_index("c")
        sid = lax.axis_index("s")
        wid = sid * NC + cid
        pltpu.sync_copy(zeros_h, acc.at[pl.ds(sid * RS0, RS0)])

        @pl.when(sid == NS - 1)
        def _():
            pltpu.sync_copy(zeros_h.at[pl.ds(0, TAIL)],
                            acc.at[pl.ds(TAIL_OFF, TAIL)])
        pltpu.sync_copy(ones_h, ones_v)
        pltpu.sync_copy(dst_h.at[pl.ds(wid * _IBA, _IBA)], didx)
        plsc.subcore_barrier()

        @pl.loop(0, _IBA)
        def _(j):
            pltpu.sync_copy(ones_v, acc.at[didx.at[j]], add=True)

        plsc.subcore_barrier()

        @pl.when(cid == 0)
        def _():
            _node_copy(lambda o, s: acc.at[pl.ds(o, s)],
                       lambda o, s: dega_h.at[pl.ds(o, s)], sid)

        @pl.when(cid == 1)
        def _():
            _node_copy(lambda o, s: acc.at[pl.ds(o, s)],
                       lambda o, s: degb_h.at[pl.ds(o, s)], sid)

    return deg_kernel(dst2d, zeros8, ones8)


_IB = ER // NS


def _sc_scatter(src2d, dst2d, g_chunks):
    nch = len(g_chunks)
    pc = nch // NC
    mesh = plsc.VectorSubcoreMesh(core_axis_name="c", subcore_axis_name="s")

    @functools.partial(
        pl.kernel,
        out_type=tuple(jax.ShapeDtypeStruct((N, CK), jnp.float32) for _ in range(nch)),
        mesh=mesh,
        scratch_types=[
            pltpu.VMEM((_IB, EB), jnp.int32),
            pltpu.VMEM((_IB // 2, EB), jnp.int32),
            pltpu.VMEM((2, EB, CK), jnp.float32),
            pltpu.VMEM_SHARED((N, CK), jnp.float32),
            pltpu.SemaphoreType.DMA,
            pltpu.SemaphoreType.DMA,
        ],
    )
    def scat_kernel(src_h, dst_h, *rest):
        gs = rest[:nch]
        outs = rest[nch:2 * nch]
        sidx, didx, rows, acc, sem0, sem1 = rest[2 * nch:]
        sems = (sem0, sem1)
        cid = lax.axis_index("c")
        sid = lax.axis_index("s")
        hp = _IB // 2
        pltpu.sync_copy(src_h.at[pl.ds(sid * _IB, _IB)], sidx)

        def do_chunk(g_h, out_h):
            _node_copy(lambda o, s: g_h.at[pl.ds(o, s)],
                       lambda o, s: acc.at[pl.ds(o, s)], sid)
            plsc.subcore_barrier()

            def gather_batch(g_h, j, b):
                pltpu.async_copy(g_h.at[sidx.at[j]], rows.at[b], sems[b])

            def wait_batch(g_h, b):
                pltpu.make_async_copy(g_h.at[sidx.at[0]], rows.at[b],
                                      sems[b]).wait()

            for b in range(2):
                gather_batch(g_h, b, b)
            for p in range(2):
                pltpu.sync_copy(dst_h.at[pl.ds(sid * _IB + p * hp, hp)], didx)

                @pl.loop(0, hp // 2)
                def _(t):
                    j0 = 2 * t
                    for b in range(2):
                        wait_batch(g_h, b)
                        pltpu.sync_copy(rows.at[b], acc.at[didx.at[j0 + b]],
                                        add=True)
                        g_next = p * hp + j0 + 2 + b

                        @pl.when(g_next < _IB)
                        def _():
                            gather_batch(g_h, g_next, b)

            plsc.subcore_barrier()
            _node_copy(lambda o, s: acc.at[pl.ds(o, s)],
                       lambda o, s: out_h.at[pl.ds(o, s)], sid)
            plsc.subcore_barrier()

        @pl.when(cid == 0)
        def _():
            for k in range(pc):
                do_chunk(gs[k], outs[k])

        @pl.when(cid == 1)
        def _():
            for k in range(pc):
                do_chunk(gs[pc + k], outs[pc + k])

    return scat_kernel(src2d, dst2d, *g_chunks)


def _tc_pre(x, w1t, dega, degb):
    nb = N // RB

    def body(x_r, w_r, da_r, db_r, dinv_r, g0_r, g1_r, g2_r, g3_r):
        d = lax.rsqrt(da_r[...] + db_r[...] + 1.0)
        dinv_r[...] = d[:, 0:8]
        h = jnp.dot(x_r[...], w_r[...], preferred_element_type=jnp.float32)
        g = h * d[:, 0:1]
        g0_r[...] = g[:, 0 * CK:1 * CK]
        g1_r[...] = g[:, 1 * CK:2 * CK]
        g2_r[...] = g[:, 2 * CK:3 * CK]
        g3_r[...] = g[:, 3 * CK:4 * CK]

    outs = pl.pallas_call(
        body,
        grid=(nb,),
        in_specs=[
            pl.BlockSpec((RB, 256), lambda i: (i, 0)),
            pl.BlockSpec((256, 512), lambda i: (0, 0)),
            pl.BlockSpec((RB, CK), lambda i: (i, 0)),
            pl.BlockSpec((RB, CK), lambda i: (i, 0)),
        ],
        out_specs=[pl.BlockSpec((RB, 8), lambda i: (i, 0))]
        + [pl.BlockSpec((RB, CK), lambda i: (i, 0)) for _ in range(4)],
        out_shape=[jax.ShapeDtypeStruct((N, 8), jnp.float32)]
        + [jax.ShapeDtypeStruct((N, CK), jnp.float32) for _ in range(4)],
    )(x, w1t, dega, degb)
    return outs[0], outs[1:]


def _tc_mid(a_chunks, dinv, b1, w2t):
    nb = N // RB

    def body(a0_r, a1_r, a2_r, a3_r, dinv_r, b_r, w_r, g0_r, g1_r):
        d = dinv_r[:, 0:1]
        h = jnp.zeros((RB, 256), jnp.float32)
        for k, a_r in enumerate((a0_r, a1_r, a2_r, a3_r)):
            z = jnp.maximum(a_r[...] * d + b_r[0:1, k * CK:(k + 1) * CK], 0.0)
            h = h + jnp.dot(z, w_r[pl.ds(k * CK, CK), :],
                            preferred_element_type=jnp.float32)
        g = h * d
        g0_r[...] = g[:, 0:CK]
        g1_r[...] = g[:, CK:2 * CK]

    outs = pl.pallas_call(
        body,
        grid=(nb,),
        in_specs=[pl.BlockSpec((RB, CK), lambda i: (i, 0)) for _ in range(4)]
        + [
            pl.BlockSpec((RB, 8), lambda i: (i, 0)),
            pl.BlockSpec((1, 512), lambda i: (0, 0)),
            pl.BlockSpec((512, 256), lambda i: (0, 0)),
        ],
        out_specs=[pl.BlockSpec((RB, CK), lambda i: (i, 0)) for _ in range(2)],
        out_shape=[jax.ShapeDtypeStruct((N, CK), jnp.float32) for _ in range(2)],
    )(*a_chunks, dinv, b1, w2t)
    return outs


def _tc_post(c_chunks, dinv, b2, wsst):
    nb = N // RB

    def body(c0_r, c1_r, dinv_r, b_r, w_r, o_r):
        d = dinv_r[:, 0:1]
        o = jnp.zeros((RB, 8), jnp.float32)
        for k, c_r in enumerate((c0_r, c1_r)):
            z = jnp.maximum(c_r[...] * d + b_r[0:1, k * CK:(k + 1) * CK], 0.0)
            o = o + jnp.dot(z, w_r[pl.ds(k * CK, CK), :],
                            preferred_element_type=jnp.float32)
        o_r[...] = o

    return pl.pallas_call(
        body,
        grid=(nb,),
        in_specs=[pl.BlockSpec((RB, CK), lambda i: (i, 0)) for _ in range(2)]
        + [
            pl.BlockSpec((RB, 8), lambda i: (i, 0)),
            pl.BlockSpec((1, 256), lambda i: (0, 0)),
            pl.BlockSpec((256, 8), lambda i: (0, 0)),
        ],
        out_specs=pl.BlockSpec((RB, 8), lambda i: (i, 0)),
        out_shape=jax.ShapeDtypeStruct((N, 8), jnp.float32),
    )(*c_chunks, dinv, b2, wsst)


def kernel(adj, x, W1, b1, W2, b2, Wss):
    src = adj[0].astype(jnp.int32)
    dst = adj[1].astype(jnp.int32)
    src2d = src.reshape(ER, EB)
    dst2d = dst.reshape(ER, EB)

    zeros8 = jnp.zeros((RS0, CK), jnp.float32)
    ones8 = jnp.ones((EB, CK), jnp.float32)


    dega, degb = _sc_degree(dst2d, zeros8, ones8)
    dinv, g1 = _tc_pre(x, W1.T, dega, degb)
    a1 = _sc_scatter(src2d, dst2d, list(g1))
    g2 = _tc_mid(a1, dinv, b1.reshape(1, 512), W2.T)
    a2 = _sc_scatter(src2d, dst2d, list(g2))
    out = _tc_post(a2, dinv, b2.reshape(1, 256), Wss.T)
    return (out, out)


# speedup vs baseline: 1.0021x; 1.0021x over previous
import functools

import jax
import jax.numpy as jnp
from jax import lax
from jax.experimental import pallas as pl
from jax.experimental.pallas import tpu as pltpu
from jax.experimental.pallas import tpu_sc as plsc

N = 10000
E = 160000
CK = 128
NC = 2
NS = 16
EB = 125
ER = E // EB
RB = 2000

RS0 = 624
TAIL = N - NS * RS0
TAIL_OFF = N - TAIL


def _node_copy(make_src, make_dst, sid):
    pltpu.sync_copy(make_src(sid * RS0, RS0), make_dst(sid * RS0, RS0))

    @pl.when(sid == NS - 1)
    def _():
        pltpu.sync_copy(make_src(TAIL_OFF, TAIL), make_dst(TAIL_OFF, TAIL))


_IBA = ER // (NC * NS)


def _sc_degree(dst2d, zeros8, ones8):
    mesh = plsc.VectorSubcoreMesh(core_axis_name="c", subcore_axis_name="s")

    @functools.partial(
        pl.kernel,
        out_type=(
            jax.ShapeDtypeStruct((N, CK), jnp.float32),
            jax.ShapeDtypeStruct((N, CK), jnp.float32),
        ),
        mesh=mesh,
        compiler_params=pltpu.CompilerParams(use_tc_tiling_on_sc=False),
        scratch_types=[
            pltpu.VMEM((_IBA, EB), jnp.int32),
            pltpu.VMEM((EB, CK), jnp.float32),
            pltpu.VMEM_SHARED((N, CK), jnp.float32),
        ],
    )
    def deg_kernel(dst_h, zeros_h, ones_h, dega_h, degb_h, didx, ones_v, acc):
        cid = lax.axis_index("c")
        sid = lax.axis_index("s")
        wid = sid * NC + cid
        pltpu.sync_copy(zeros_h, acc.at[pl.ds(sid * RS0, RS0)])

        @pl.when(sid == NS - 1)
        def _():
            pltpu.sync_copy(zeros_h.at[pl.ds(0, TAIL)],
                            acc.at[pl.ds(TAIL_OFF, TAIL)])
        pltpu.sync_copy(ones_h, ones_v)
        pltpu.sync_copy(dst_h.at[pl.ds(wid * _IBA, _IBA)], didx)
        plsc.subcore_barrier()

        @pl.loop(0, _IBA)
        def _(j):
            pltpu.sync_copy(ones_v, acc.at[didx.at[j]], add=True)

        plsc.subcore_barrier()

        @pl.when(cid == 0)
        def _():
            _node_copy(lambda o, s: acc.at[pl.ds(o, s)],
                       lambda o, s: dega_h.at[pl.ds(o, s)], sid)

        @pl.when(cid == 1)
        def _():
            _node_copy(lambda o, s: acc.at[pl.ds(o, s)],
                       lambda o, s: degb_h.at[pl.ds(o, s)], sid)

    return deg_kernel(dst2d, zeros8, ones8)


_IB = ER // NS


def _sc_scatter(src2d, dst2d, g_chunks):
    nch = len(g_chunks)
    pc = nch // NC
    mesh = plsc.VectorSubcoreMesh(core_axis_name="c", subcore_axis_name="s")

    @functools.partial(
        pl.kernel,
        out_type=tuple(jax.ShapeDtypeStruct((N, CK), jnp.float32) for _ in range(nch)),
        mesh=mesh,
        compiler_params=pltpu.CompilerParams(use_tc_tiling_on_sc=False),
        scratch_types=[
            pltpu.VMEM((_IB, EB), jnp.int32),
            pltpu.VMEM((_IB // 2, EB), jnp.int32),
            pltpu.VMEM((2, EB, CK), jnp.float32),
            pltpu.VMEM_SHARED((N, CK), jnp.float32),
            pltpu.SemaphoreType.DMA,
            pltpu.SemaphoreType.DMA,
        ],
    )
    def scat_kernel(src_h, dst_h, *rest):
        gs = rest[:nch]
        outs = rest[nch:2 * nch]
        sidx, didx, rows, acc, sem0, sem1 = rest[2 * nch:]
        sems = (sem0, sem1)
        cid = lax.axis_index("c")
        sid = lax.axis_index("s")
        hp = _IB // 2
        pltpu.sync_copy(src_h.at[pl.ds(sid * _IB, _IB)], sidx)

        def do_chunk(g_h, out_h):
            _node_copy(lambda o, s: g_h.at[pl.ds(o, s)],
                       lambda o, s: acc.at[pl.ds(o, s)], sid)
            plsc.subcore_barrier()

            def gather_batch(g_h, j, b):
                pltpu.async_copy(g_h.at[sidx.at[j]], rows.at[b], sems[b])

            def wait_batch(g_h, b):
                pltpu.make_async_copy(g_h.at[sidx.at[0]], rows.at[b],
                                      sems[b]).wait()

            for b in range(2):
                gather_batch(g_h, b, b)
            for p in range(2):
                pltpu.sync_copy(dst_h.at[pl.ds(sid * _IB + p * hp, hp)], didx)

                @pl.loop(0, hp // 2)
                def _(t):
                    j0 = 2 * t
                    for b in range(2):
                        wait_batch(g_h, b)
                        pltpu.sync_copy(rows.at[b], acc.at[didx.at[j0 + b]],
                                        add=True)
                        g_next = p * hp + j0 + 2 + b

                        @pl.when(g_next < _IB)
                        def _():
                            gather_batch(g_h, g_next, b)

            plsc.subcore_barrier()
            _node_copy(lambda o, s: acc.at[pl.ds(o, s)],
                       lambda o, s: out_h.at[pl.ds(o, s)], sid)
            plsc.subcore_barrier()

        @pl.when(cid == 0)
        def _():
            for k in range(pc):
                do_chunk(gs[k], outs[k])

        @pl.when(cid == 1)
        def _():
            for k in range(pc):
                do_chunk(gs[pc + k], outs[pc + k])

    return scat_kernel(src2d, dst2d, *g_chunks)


def _tc_pre(x, w1t, dega, degb):
    nb = N // RB

    def body(x_r, w_r, da_r, db_r, dinv_r, g0_r, g1_r, g2_r, g3_r):
        d = lax.rsqrt(da_r[...] + db_r[...] + 1.0)
        dinv_r[...] = d[:, 0:8]
        h = jnp.dot(x_r[...], w_r[...], preferred_element_type=jnp.float32)
        g = h * d[:, 0:1]
        g0_r[...] = g[:, 0 * CK:1 * CK]
        g1_r[...] = g[:, 1 * CK:2 * CK]
        g2_r[...] = g[:, 2 * CK:3 * CK]
        g3_r[...] = g[:, 3 * CK:4 * CK]

    outs = pl.pallas_call(
        body,
        grid=(nb,),
        in_specs=[
            pl.BlockSpec((RB, 256), lambda i: (i, 0)),
            pl.BlockSpec((256, 512), lambda i: (0, 0)),
            pl.BlockSpec((RB, CK), lambda i: (i, 0)),
            pl.BlockSpec((RB, CK), lambda i: (i, 0)),
        ],
        out_specs=[pl.BlockSpec((RB, 8), lambda i: (i, 0))]
        + [pl.BlockSpec((RB, CK), lambda i: (i, 0)) for _ in range(4)],
        out_shape=[jax.ShapeDtypeStruct((N, 8), jnp.float32)]
        + [jax.ShapeDtypeStruct((N, CK), jnp.float32) for _ in range(4)],
    )(x, w1t, dega, degb)
    return outs[0], outs[1:]


def _tc_mid(a_chunks, dinv, b1, w2t):
    nb = N // RB

    def body(a0_r, a1_r, a2_r, a3_r, dinv_r, b_r, w_r, g0_r, g1_r):
        d = dinv_r[:, 0:1]
        h = jnp.zeros((RB, 256), jnp.float32)
        for k, a_r in enumerate((a0_r, a1_r, a2_r, a3_r)):
            z = jnp.maximum(a_r[...] * d + b_r[0:1, k * CK:(k + 1) * CK], 0.0)
            h = h + jnp.dot(z, w_r[pl.ds(k * CK, CK), :],
                            preferred_element_type=jnp.float32)
        g = h * d
        g0_r[...] = g[:, 0:CK]
        g1_r[...] = g[:, CK:2 * CK]

    outs = pl.pallas_call(
        body,
        grid=(nb,),
        in_specs=[pl.BlockSpec((RB, CK), lambda i: (i, 0)) for _ in range(4)]
        + [
            pl.BlockSpec((RB, 8), lambda i: (i, 0)),
            pl.BlockSpec((1, 512), lambda i: (0, 0)),
            pl.BlockSpec((512, 256), lambda i: (0, 0)),
        ],
        out_specs=[pl.BlockSpec((RB, CK), lambda i: (i, 0)) for _ in range(2)],
        out_shape=[jax.ShapeDtypeStruct((N, CK), jnp.float32) for _ in range(2)],
    )(*a_chunks, dinv, b1, w2t)
    return outs


def _tc_post(c_chunks, dinv, b2, wsst):
    nb = N // RB

    def body(c0_r, c1_r, dinv_r, b_r, w_r, o_r):
        d = dinv_r[:, 0:1]
        o = jnp.zeros((RB, 8), jnp.float32)
        for k, c_r in enumerate((c0_r, c1_r)):
            z = jnp.maximum(c_r[...] * d + b_r[0:1, k * CK:(k + 1) * CK], 0.0)
            o = o + jnp.dot(z, w_r[pl.ds(k * CK, CK), :],
                            preferred_element_type=jnp.float32)
        o_r[...] = o

    return pl.pallas_call(
        body,
        grid=(nb,),
        in_specs=[pl.BlockSpec((RB, CK), lambda i: (i, 0)) for _ in range(2)]
        + [
            pl.BlockSpec((RB, 8), lambda i: (i, 0)),
            pl.BlockSpec((1, 256), lambda i: (0, 0)),
            pl.BlockSpec((256, 8), lambda i: (0, 0)),
        ],
        out_specs=pl.BlockSpec((RB, 8), lambda i: (i, 0)),
        out_shape=jax.ShapeDtypeStruct((N, 8), jnp.float32),
    )(*c_chunks, dinv, b2, wsst)


def kernel(adj, x, W1, b1, W2, b2, Wss):
    src = adj[0].astype(jnp.int32)
    dst = adj[1].astype(jnp.int32)
    src2d = src.reshape(ER, EB)
    dst2d = dst.reshape(ER, EB)

    zeros8 = jnp.zeros((RS0, CK), jnp.float32)
    ones8 = jnp.ones((EB, CK), jnp.float32)


    dega, degb = _sc_degree(dst2d, zeros8, ones8)
    dinv, g1 = _tc_pre(x, W1.T, dega, degb)
    a1 = _sc_scatter(src2d, dst2d, list(g1))
    g2 = _tc_mid(a1, dinv, b1.reshape(1, 512), W2.T)
    a2 = _sc_scatter(src2d, dst2d, list(g2))
    out = _tc_post(a2, dinv, b2.reshape(1, 256), Wss.T)
    return (out, out)
